# baseline (speedup 1.0000x reference)
import functools

import jax
import jax.numpy as jnp
from jax import lax
from jax.experimental import pallas as pl
from jax.experimental.pallas import tpu as pltpu
from jax.experimental.pallas import tpu_sc as plsc

N = 10000
NPAD = 10240
D = 128
E_EDGES = 160000

_NC = 2
_NS = 16
_L = 16

_EPT = E_EDGES // _NS
_EPB = 40
_NB = _EPT // _EPB
_NSLOT = 5
_RPT = NPAD // _NS


def _seg_body(item_hbm, user_hbm, src_hbm, dst_hbm, ev_hbm,
              aggu_hbm, aggi_hbm,
              acc, gidx_v, ev_v,
              sidx0, sidx1, sidx2, sidx3, sidx4,
              rows0, rows1, rows2, rows3, rows4,
              frows,
              semg0, semg1, semg2, semg3, semg4,
              sems0, sems1, sems2, sems3, sems4):
    c = lax.axis_index("c")
    s = lax.axis_index("s")
    sidx = (sidx0, sidx1, sidx2, sidx3, sidx4)
    rows = (rows0, rows1, rows2, rows3, rows4)
    semg = (semg0, semg1, semg2, semg3, semg4)
    sems = (sems0, sems1, sems2, sems3, sems4)
    zeros16 = jnp.zeros((_L,), jnp.float32)

    def _run(table_hbm, g_hbm, s_hbm, out_hbm):
        base_t = s * _EPT
        def _z(e, _):
            for ch in range(D // _L):
                frows[e, pl.ds(ch * _L, _L)] = zeros16
            return 0
        lax.fori_loop(0, _EPB, _z, 0)
        for j in range(_RPT // _EPB):
            pltpu.sync_copy(frows, acc.at[pl.ds(s * _RPT + j * _EPB, _EPB)])
        pltpu.sync_copy(g_hbm.at[pl.ds(base_t, _EPT)], gidx_v)
        pltpu.sync_copy(ev_hbm.at[pl.ds(base_t, _EPT)], ev_v)
        plsc.subcore_barrier()

        def _prefetch(b, k):
            pltpu.async_copy(s_hbm.at[pl.ds(base_t + b * _EPB, _EPB)],
                             sidx[k], sems[k])
            pltpu.async_copy(table_hbm.at[gidx_v.at[pl.ds(b * _EPB, _EPB)]],
                             rows[k], semg[k])

        for k in range(_NSLOT):
            _prefetch(k, k)

        def _outer(i, _):
            for k in range(_NSLOT):
                b = i * _NSLOT + k
                pltpu.make_async_copy(table_hbm.at[gidx_v.at[pl.ds(0, _EPB)]],
                                      rows[k], semg[k]).wait()
                def _scale(e4, _, _k=k):
                    for de in range(4):
                        e = e4 * 4 + de
                        evb = plsc.load_gather(
                            ev_v, [jnp.full((_L,), b * _EPB + e, jnp.int32)])
                        for ch in range(D // 32):
                            w32 = rows[_k][e, pl.ds(ch * _L, _L)]
                            vb = plsc.bitcast(w32, jnp.bfloat16)
                            ae, ao = plsc.unpack(
                                vb, format=plsc.PackFormat.INTERLEAVED)
                            frows[e, pl.ds(ch * 32, _L)] = ae * evb
                            frows[e, pl.ds(ch * 32 + _L, _L)] = ao * evb
                    return 0
                lax.fori_loop(0, _EPB // 4, _scale, 0)
                pltpu.make_async_copy(s_hbm.at[pl.ds(0, _EPB)],
                                      sidx[k], sems[k]).wait()
                pltpu.sync_copy(frows, acc.at[sidx[k]], add=True)

                @pl.when(b + _NSLOT < _NB)
                def _():
                    _prefetch(b + _NSLOT, k)
            return 0

        lax.fori_loop(0, _NB // _NSLOT, _outer, 0)
        plsc.subcore_barrier()
        pltpu.sync_copy(acc.at[pl.ds(s * _RPT, _RPT)],
                        out_hbm.at[pl.ds(s * _RPT, _RPT)])

    @pl.when(c == 0)
    def _():
        _run(item_hbm, dst_hbm, src_hbm, aggu_hbm)

    @pl.when(c == 1)
    def _():
        _run(user_hbm, src_hbm, dst_hbm, aggi_hbm)


def _seg_sums(item_pk, user_pk, src, dst, ev):
    sd = jax.ShapeDtypeStruct((NPAD, D), jnp.float32)
    mesh = plsc.VectorSubcoreMesh(core_axis_name="c", subcore_axis_name="s",
                                  num_cores=_NC, num_subcores=_NS)
    f = pl.kernel(
        _seg_body,
        out_type=(sd, sd),
        mesh=mesh,
        compiler_params=pltpu.CompilerParams(needs_layout_passes=False,
                                             use_tc_tiling_on_sc=False),
        scratch_types=(
            [pltpu.VMEM_SHARED((NPAD, D), jnp.float32),
             pltpu.VMEM((_EPT,), jnp.int32),
             pltpu.VMEM((_EPT,), jnp.float32)]
            + [pltpu.VMEM((_EPB,), jnp.int32) for _ in range(_NSLOT)]
            + [pltpu.VMEM((_EPB, D // 2), jnp.int32) for _ in range(_NSLOT)]
            + [pltpu.VMEM((_EPB, D), jnp.float32)]
            + [pltpu.SemaphoreType.DMA for _ in range(2 * _NSLOT)]
        ),
    )
    return f(item_pk, user_pk, src, dst, ev)


def _attn_body(aggu_ref, aggi_ref, item_ref, w_ref, wp_ref, o_ref,
               k_scr, v_scr):
    wb = w_ref[...].astype(jnp.bfloat16)
    wpb = wp_ref[...].astype(jnp.bfloat16)

    @pl.when(pl.program_id(0) == 0)
    def _():
        k_scr[...] = jax.lax.dot_general(
            aggi_ref[...].astype(jnp.bfloat16), wpb, (((1,), (0,)), ((), ())),
            preferred_element_type=jnp.float32).astype(jnp.bfloat16)
        v_scr[...] = jax.lax.dot_general(
            item_ref[...].astype(jnp.bfloat16), wb, (((1,), (0,)), ((), ())),
            preferred_element_type=jnp.float32).astype(jnp.bfloat16)

    q = jax.lax.dot_general(
        aggu_ref[...].astype(jnp.bfloat16), wpb, (((1,), (0,)), ((), ())),
        preferred_element_type=jnp.float32).astype(jnp.bfloat16)
    s = jax.lax.dot_general(
        q, k_scr[...], (((1,), (1,)), ((), ())),
        preferred_element_type=jnp.float32)
    p = jnp.exp(s).astype(jnp.bfloat16)
    l = jnp.sum(p, axis=1, keepdims=True, dtype=jnp.float32)
    l = l - jnp.float32(NPAD - N)
    o = jax.lax.dot_general(
        p, v_scr[...], (((1,), (0,)), ((), ())),
        preferred_element_type=jnp.float32)
    o = o / l
    o_ref[...] = jnp.dot(o, w_ref[...], preferred_element_type=jnp.float32)


def _attn(agg_u, agg_i, item_pad, w, w_perm):
    bq = 512
    grid = (NPAD // bq,)
    return pl.pallas_call(
        _attn_body,
        grid=grid,
        in_specs=[
            pl.BlockSpec((bq, D), lambda i: (i, 0)),
            pl.BlockSpec((NPAD, D), lambda i: (0, 0)),
            pl.BlockSpec((NPAD, D), lambda i: (0, 0)),
            pl.BlockSpec((D, D), lambda i: (0, 0)),
            pl.BlockSpec((D, D), lambda i: (0, 0)),
        ],
        out_specs=pl.BlockSpec((bq, D), lambda i: (i, 0)),
        out_shape=jax.ShapeDtypeStruct((NPAD, D), jnp.float32),
        scratch_shapes=[
            pltpu.VMEM((NPAD, D), jnp.bfloat16),
            pltpu.VMEM((NPAD, D), jnp.bfloat16),
        ],
    )(agg_u, agg_i, item_pad, w, w_perm)


def kernel(user_emb, item_emb, attention_weight, edge_index, edge_values):
    src = edge_index[0].astype(jnp.int32)
    dst = edge_index[1].astype(jnp.int32)
    ev = edge_values

    item_pad = jnp.pad(item_emb, ((0, NPAD - N), (0, 0)))
    item_pk = jax.lax.bitcast_convert_type(
        item_emb.astype(jnp.bfloat16).reshape(N, D // 2, 2), jnp.int32)
    user_pk = jax.lax.bitcast_convert_type(
        user_emb.astype(jnp.bfloat16).reshape(N, D // 2, 2), jnp.int32)
    item_pk = jnp.pad(item_pk, ((0, NPAD - N), (0, 0)))
    user_pk = jnp.pad(user_pk, ((0, NPAD - N), (0, 0)))

    agg_u, agg_i = _seg_sums(item_pk, user_pk, src, dst, ev)

    sig32 = jnp.concatenate([jnp.arange(0, 32, 2), jnp.arange(1, 32, 2)])
    sigma = (jnp.arange(D // 32)[:, None] * 32 + sig32[None, :]).reshape(-1)
    w_perm = attention_weight[sigma, :]

    out = _attn(agg_u, agg_i, item_pad, attention_weight, w_perm)
    return out[:N]


# speedup vs baseline: 1.6059x; 1.6059x over previous
import jax
import jax.numpy as jnp
from jax import lax
from jax.experimental import pallas as pl
from jax.experimental.pallas import tpu as pltpu
from jax.experimental.pallas import tpu_sc as plsc

N = 10000
NPAD = 10240
D = 128
E_EDGES = 160000

_NC = 2
_NS = 16
_L = 16

_EPT = E_EDGES // _NS
_EPB = 40
_NB = _EPT // _EPB
_NSLOT = 5
_RPT = NPAD // _NS


def _seg_body(item_hbm, user_hbm, src_hbm, dst_hbm, ev_hbm,
              aggu_hbm, aggi_hbm,
              acc, gidx_v, ev_v,
              sidx0, sidx1, sidx2, sidx3, sidx4,
              rows0, rows1, rows2, rows3, rows4,
              semg0, semg1, semg2, semg3, semg4,
              sems0, sems1, sems2, sems3, sems4):
    c = lax.axis_index("c")
    s = lax.axis_index("s")
    sidx = (sidx0, sidx1, sidx2, sidx3, sidx4)
    rows = (rows0, rows1, rows2, rows3, rows4)
    semg = (semg0, semg1, semg2, semg3, semg4)
    sems = (sems0, sems1, sems2, sems3, sems4)
    zeros16 = jnp.zeros((_L,), jnp.float32)

    def _run(table_hbm, g_hbm, s_hbm, out_hbm):
        base_t = s * _EPT
        def _z(e, _):
            for ch in range(D // _L):
                rows0[e, pl.ds(ch * _L, _L)] = zeros16
            return 0
        lax.fori_loop(0, _EPB, _z, 0)
        for j in range(_RPT // _EPB):
            pltpu.sync_copy(rows0, acc.at[pl.ds(s * _RPT + j * _EPB, _EPB)])
        pltpu.sync_copy(g_hbm.at[pl.ds(base_t, _EPT)], gidx_v)
        pltpu.sync_copy(ev_hbm.at[pl.ds(base_t, _EPT)], ev_v)
        plsc.subcore_barrier()

        def _prefetch(b, k):
            pltpu.async_copy(s_hbm.at[pl.ds(base_t + b * _EPB, _EPB)],
                             sidx[k], sems[k])
            pltpu.async_copy(table_hbm.at[gidx_v.at[pl.ds(b * _EPB, _EPB)]],
                             rows[k], semg[k])

        for k in range(_NSLOT):
            _prefetch(k, k)

        def _outer(i, _):
            for k in range(_NSLOT):
                b = i * _NSLOT + k
                pltpu.make_async_copy(table_hbm.at[gidx_v.at[pl.ds(0, _EPB)]],
                                      rows[k], semg[k]).wait()
                def _scale(e4, _, _k=k):
                    for de in range(4):
                        e = e4 * 4 + de
                        evb = plsc.load_gather(
                            ev_v, [jnp.full((_L,), b * _EPB + e, jnp.int32)])
                        for ch in range(D // _L):
                            sl = (e, pl.ds(ch * _L, _L))
                            rows[_k][sl] = rows[_k][sl] * evb
                    return 0
                lax.fori_loop(0, _EPB // 4, _scale, 0)
                pltpu.make_async_copy(s_hbm.at[pl.ds(0, _EPB)],
                                      sidx[k], sems[k]).wait()
                pltpu.sync_copy(rows[k], acc.at[sidx[k]], add=True)

                @pl.when(b + _NSLOT < _NB)
                def _():
                    _prefetch(b + _NSLOT, k)
            return 0

        lax.fori_loop(0, _NB // _NSLOT, _outer, 0)
        plsc.subcore_barrier()
        pltpu.sync_copy(acc.at[pl.ds(s * _RPT, _RPT)],
                        out_hbm.at[pl.ds(s * _RPT, _RPT)])

    @pl.when(c == 0)
    def _():
        _run(item_hbm, dst_hbm, src_hbm, aggu_hbm)

    @pl.when(c == 1)
    def _():
        _run(user_hbm, src_hbm, dst_hbm, aggi_hbm)


def _seg_sums(item_pad, user_pad, src, dst, ev):
    sd = jax.ShapeDtypeStruct((NPAD, D), jnp.float32)
    mesh = plsc.VectorSubcoreMesh(core_axis_name="c", subcore_axis_name="s",
                                  num_cores=_NC, num_subcores=_NS)
    f = pl.kernel(
        _seg_body,
        out_type=(sd, sd),
        mesh=mesh,
        compiler_params=pltpu.CompilerParams(needs_layout_passes=False),
        scratch_types=(
            [pltpu.VMEM_SHARED((NPAD, D), jnp.float32),
             pltpu.VMEM((_EPT,), jnp.int32),
             pltpu.VMEM((_EPT,), jnp.float32)]
            + [pltpu.VMEM((_EPB,), jnp.int32) for _ in range(_NSLOT)]
            + [pltpu.VMEM((_EPB, D), jnp.float32) for _ in range(_NSLOT)]
            + [pltpu.SemaphoreType.DMA for _ in range(2 * _NSLOT)]
        ),
    )
    return f(item_pad, user_pad, src, dst, ev)


def _attn_body(aggu_ref, aggi_ref, item_ref, w_ref, o_ref, k_scr, v_scr):
    wb = w_ref[...].astype(jnp.bfloat16)

    @pl.when(pl.program_id(0) == 0)
    def _():
        k_scr[...] = jax.lax.dot_general(
            aggi_ref[...].astype(jnp.bfloat16), wb, (((1,), (0,)), ((), ())),
            preferred_element_type=jnp.float32).astype(jnp.bfloat16)
        v_scr[...] = jax.lax.dot_general(
            item_ref[...].astype(jnp.bfloat16), wb, (((1,), (0,)), ((), ())),
            preferred_element_type=jnp.float32).astype(jnp.bfloat16)

    q = jax.lax.dot_general(
        aggu_ref[...].astype(jnp.bfloat16), wb, (((1,), (0,)), ((), ())),
        preferred_element_type=jnp.float32).astype(jnp.bfloat16)
    s = jax.lax.dot_general(
        q, k_scr[...], (((1,), (1,)), ((), ())),
        preferred_element_type=jnp.float32)
    p = jnp.exp(s).astype(jnp.bfloat16)
    l = jnp.sum(p, axis=1, keepdims=True, dtype=jnp.float32)
    l = l - jnp.float32(NPAD - N)
    o = jax.lax.dot_general(
        p, v_scr[...], (((1,), (0,)), ((), ())),
        preferred_element_type=jnp.float32)
    o = o / l
    o_ref[...] = jnp.dot(o, w_ref[...], preferred_element_type=jnp.float32)


def _attn(agg_u, agg_i, item_pad, w):
    bq = 512
    grid = (NPAD // bq,)
    return pl.pallas_call(
        _attn_body,
        grid=grid,
        in_specs=[
            pl.BlockSpec((bq, D), lambda i: (i, 0)),
            pl.BlockSpec((NPAD, D), lambda i: (0, 0)),
            pl.BlockSpec((NPAD, D), lambda i: (0, 0)),
            pl.BlockSpec((D, D), lambda i: (0, 0)),
        ],
        out_specs=pl.BlockSpec((bq, D), lambda i: (i, 0)),
        out_shape=jax.ShapeDtypeStruct((NPAD, D), jnp.float32),
        scratch_shapes=[
            pltpu.VMEM((NPAD, D), jnp.bfloat16),
            pltpu.VMEM((NPAD, D), jnp.bfloat16),
        ],
    )(agg_u, agg_i, item_pad, w)


def kernel(user_emb, item_emb, attention_weight, edge_index, edge_values):
    src = edge_index[0].astype(jnp.int32)
    dst = edge_index[1].astype(jnp.int32)
    ev = edge_values

    user_pad = jnp.pad(user_emb, ((0, NPAD - N), (0, 0)))
    item_pad = jnp.pad(item_emb, ((0, NPAD - N), (0, 0)))

    agg_u, agg_i = _seg_sums(item_pad, user_pad, src, dst, ev)

    out = _attn(agg_u, agg_i, item_pad, attention_weight)
    return out[:N]
